# baseline (speedup 1.0000x reference)
import functools

import jax
import jax.numpy as jnp
from jax import lax
from jax.experimental import pallas as pl
from jax.experimental.pallas import tpu as pltpu
from jax.experimental.pallas import tpu_sc as plsc

N = 10000
E = 320000
D = 128

NC = 2
NS = 16
LANES = 16
C = 40
EPT = E // (NC * NS)
NCHUNK = EPT // C
ROW_BLK = 1000


def _bf16_bits(y):
    yb = y.astype(jnp.bfloat16).astype(jnp.float32)
    return lax.shift_right_logical(lax.bitcast_convert_type(yb, jnp.int32), 16)


def _mm_body(x_ref, wk, bk, wqe, bqe, wqo, bqo, wve, bve, wvo, bvo, ws, bs,
             k_out, qv_out, b_out):
    xb = x_ref[...]
    k_out[...] = -(jnp.dot(xb, wk[...], preferred_element_type=jnp.float32)
                   + bk[...])
    qe = jnp.dot(xb, wqe[...], preferred_element_type=jnp.float32) + bqe[...]
    qo = jnp.dot(xb, wqo[...], preferred_element_type=jnp.float32) + bqo[...]
    ve = jnp.dot(xb, wve[...], preferred_element_type=jnp.float32) + bve[...]
    vo = jnp.dot(xb, wvo[...], preferred_element_type=jnp.float32) + bvo[...]
    wq = jnp.bitwise_or(_bf16_bits(qe), lax.shift_left(_bf16_bits(qo), 16))
    wv = jnp.bitwise_or(_bf16_bits(ve), lax.shift_left(_bf16_bits(vo), 16))
    qv_out[...] = jnp.concatenate([wq, wv], axis=1)
    b_out[...] = jnp.dot(xb, ws[...], preferred_element_type=jnp.float32) + bs[...]


def _stage1(x, Wk, bk, Wq, bq, Wv, bv, Ws, bias):
    nblk = N // ROW_BLK
    row_spec = pl.BlockSpec((ROW_BLK, D), lambda i: (i, 0))
    w_spec = pl.BlockSpec((D, D), lambda i: (0, 0))
    h_spec = pl.BlockSpec((D, D // 2), lambda i: (0, 0))
    b_spec = pl.BlockSpec((1, D), lambda i: (0, 0))
    bh_spec = pl.BlockSpec((1, D // 2), lambda i: (0, 0))
    out_qv = jax.ShapeDtypeStruct((N, D), jnp.int32)
    out32 = jax.ShapeDtypeStruct((N, D), jnp.float32)
    lo = jnp.asarray([32 * i + r for i in range(D // 32) for r in range(16)],
                     dtype=jnp.int32)
    hi = lo + 16
    return pl.pallas_call(
        _mm_body,
        grid=(nblk,),
        in_specs=[row_spec, w_spec, b_spec, h_spec, bh_spec, h_spec, bh_spec,
                  h_spec, bh_spec, h_spec, bh_spec, w_spec, b_spec],
        out_specs=[row_spec, row_spec, row_spec],
        out_shape=[out32, out_qv, out32],
    )(x, Wk, bk.reshape(1, D),
      Wq[:, lo], bq[lo].reshape(1, D // 2), Wq[:, hi], bq[hi].reshape(1, D // 2),
      Wv[:, lo], bv[lo].reshape(1, D // 2), Wv[:, hi], bv[hi].reshape(1, D // 2),
      Ws, bias.reshape(1, D))


def _sc_body(src_hbm, dst_hbm, k_hbm, qv_hbm, base_hbm, zero_hbm,
             p0_hbm, p1_hbm,
             di0, si0, ds0, kd0, qv0, mg0, di1, si1, ds1, kd1, qv1, mg1, agg,
             semi0, semk0, semq0, sems0, semi1, semk1, semq1, sems1):
    cid = lax.axis_index("c")
    sid = lax.axis_index("s")
    wid = cid * NS + sid

    @pl.when(sid == 0)
    def _():
        @pl.when(cid == 0)
        def _():
            pltpu.sync_copy(base_hbm, agg)

        @pl.when(cid == 1)
        def _():
            pltpu.sync_copy(zero_hbm, agg)

    plsc.subcore_barrier()

    bufs = ((di0, si0, ds0, kd0, qv0, mg0, semi0, semk0, semq0, sems0),
            (di1, si1, ds1, kd1, qv1, mg1, semi1, semk1, semq1, sems1))

    def idx_start(b, t):
        di, si, dsc, kd, qv, mg, smi, sk, sq, ss = bufs[b]
        pltpu.async_copy(dst_hbm.at[wid, pl.ds(t, 1)], di, smi)
        pltpu.async_copy(dst_hbm.at[wid, pl.ds(t, 1)], dsc, smi)
        pltpu.async_copy(src_hbm.at[wid, pl.ds(t, 1)], si, smi)

    def idx_wait(b, t):
        di, si, dsc, kd, qv, mg, smi, sk, sq, ss = bufs[b]
        pltpu.make_async_copy(dst_hbm.at[wid, pl.ds(t, 1)], di, smi).wait()
        pltpu.make_async_copy(dst_hbm.at[wid, pl.ds(t, 1)], dsc, smi).wait()
        pltpu.make_async_copy(src_hbm.at[wid, pl.ds(t, 1)], si, smi).wait()

    def start_g(b, t):
        di, si, dsc, kd, qv, mg, smi, sk, sq, ss = bufs[b]
        idx_wait(b, t)
        pltpu.async_copy(k_hbm.at[di.at[0]], kd, sk)
        pltpu.async_copy(qv_hbm.at[si.at[0]], qv, sq)

    def scat_wait(b):
        di, si, dsc, kd, qv, mg, smi, sk, sq, ss = bufs[b]
        pltpu.make_async_copy(mg, agg.at[dsc.at[0]], ss).wait()

    def finish(b, t):
        di, si, dsc, kd, qv, mg, smi, sk, sq, ss = bufs[b]
        pltpu.make_async_copy(k_hbm.at[di.at[0]], kd, sk).wait()
        pltpu.make_async_copy(qv_hbm.at[si.at[0]], qv, sq).wait()

        @pl.when(t >= 2)
        def _():
            scat_wait(b)

        hi_mask = jnp.full((LANES,), -65536, dtype=jnp.int32)
        sh16 = jnp.full((LANES,), 16, dtype=jnp.int32)
        one = jnp.full((LANES,), 1.0, dtype=jnp.float32)

        def _halves(w):
            lo = lax.bitcast_convert_type(lax.shift_left(w, sh16), jnp.float32)
            hi = lax.bitcast_convert_type(lax.bitwise_and(w, hi_mask), jnp.float32)
            return lo, hi

        @plsc.parallel_loop(0, C, unroll=2)
        def _(j):
            for i in range(D // 32):
                qa, qb = _halves(qv[j, pl.ds(i * LANES, LANES)])
                va, vb = _halves(qv[j, pl.ds(D // 2 + i * LANES, LANES)])
                ka = kd[j, pl.ds(i * 32, LANES)]
                kb = kd[j, pl.ds(i * 32 + LANES, LANES)]
                mg[j, pl.ds(i * 32, LANES)] = va / (one + jnp.exp(ka - qa))
                mg[j, pl.ds(i * 32 + LANES, LANES)] = vb / (one + jnp.exp(kb - qb))
        pltpu.async_copy(mg, agg.at[dsc.at[0]], ss, add=True)

        @pl.when(t + 2 < NCHUNK)
        def _():
            idx_start(b, t + 2)

    idx_start(0, 0)
    idx_start(1, 1)
    start_g(0, 0)

    def pair_body(it, carry):
        c0 = it * 2
        c1 = c0 + 1
        start_g(1, c1)
        finish(0, c0)

        @pl.when(c1 + 1 < NCHUNK)
        def _():
            start_g(0, c1 + 1)

        finish(1, c1)
        return carry

    lax.fori_loop(0, NCHUNK // 2, pair_body, 0)

    scat_wait(0)
    scat_wait(1)

    plsc.subcore_barrier()

    rows_pt = 624
    tail = N - (NS - 1) * rows_pt
    roff = pl.multiple_of(sid * rows_pt, 8)

    @pl.when(cid == 0)
    def _():
        @pl.when(sid < NS - 1)
        def _():
            pltpu.sync_copy(agg.at[pl.ds(roff, rows_pt)],
                            p0_hbm.at[pl.ds(roff, rows_pt)])

        @pl.when(sid == NS - 1)
        def _():
            pltpu.sync_copy(agg.at[pl.ds((NS - 1) * rows_pt, tail)],
                            p0_hbm.at[pl.ds((NS - 1) * rows_pt, tail)])

    @pl.when(cid == 1)
    def _():
        @pl.when(sid < NS - 1)
        def _():
            pltpu.sync_copy(agg.at[pl.ds(roff, rows_pt)],
                            p1_hbm.at[pl.ds(roff, rows_pt)])

        @pl.when(sid == NS - 1)
        def _():
            pltpu.sync_copy(agg.at[pl.ds((NS - 1) * rows_pt, tail)],
                            p1_hbm.at[pl.ds((NS - 1) * rows_pt, tail)])


@functools.lru_cache(maxsize=1)
def _sc_edges():
    return pl.kernel(
        _sc_body,
        out_type=[jax.ShapeDtypeStruct((N, D), jnp.float32)] * 2,
        mesh=plsc.VectorSubcoreMesh(core_axis_name="c", subcore_axis_name="s",
                                    num_cores=NC, num_subcores=NS),
        scratch_types=[
            pltpu.VMEM((1, C), jnp.int32),
            pltpu.VMEM((1, C), jnp.int32),
            pltpu.VMEM((1, C), jnp.int32),
            pltpu.VMEM((C, D), jnp.float32),
            pltpu.VMEM((C, D), jnp.int32),
            pltpu.VMEM((C, D), jnp.float32),
            pltpu.VMEM((1, C), jnp.int32),
            pltpu.VMEM((1, C), jnp.int32),
            pltpu.VMEM((1, C), jnp.int32),
            pltpu.VMEM((C, D), jnp.float32),
            pltpu.VMEM((C, D), jnp.int32),
            pltpu.VMEM((C, D), jnp.float32),
            pltpu.VMEM_SHARED((N, D), jnp.float32),
            pltpu.SemaphoreType.DMA,
            pltpu.SemaphoreType.DMA,
            pltpu.SemaphoreType.DMA,
            pltpu.SemaphoreType.DMA,
            pltpu.SemaphoreType.DMA,
            pltpu.SemaphoreType.DMA,
            pltpu.SemaphoreType.DMA,
            pltpu.SemaphoreType.DMA,
        ],
    )


def _add_body(a_ref, b_ref, o_ref):
    o_ref[...] = a_ref[...] + b_ref[...]


def _stage3(p0, p1):
    row_spec = pl.BlockSpec((ROW_BLK, D), lambda i: (i, 0))
    return pl.pallas_call(
        _add_body,
        grid=(N // ROW_BLK,),
        in_specs=[row_spec, row_spec],
        out_specs=row_spec,
        out_shape=jax.ShapeDtypeStruct((N, D), jnp.float32),
    )(p0, p1)


def kernel(x, edge_index, Wk, bk, Wq, bq, Wv, bv, Ws, bias):
    src = edge_index[0].reshape(NC * NS, NCHUNK, C)
    dst = edge_index[1].reshape(NC * NS, NCHUNK, C)
    kneg, qv32, base = _stage1(x, Wk, bk, Wq, bq, Wv, bv, Ws, bias)
    zeros = jnp.zeros((N, D), dtype=jnp.float32)
    p0, p1 = _sc_edges()(src, dst, kneg, qv32, base, zeros)
    return _stage3(p0, p1)


# speedup vs baseline: 1.0091x; 1.0091x over previous
import functools

import jax
import jax.numpy as jnp
from jax import lax
from jax.experimental import pallas as pl
from jax.experimental.pallas import tpu as pltpu
from jax.experimental.pallas import tpu_sc as plsc

N = 10000
E = 320000
D = 128

NC = 2
NS = 16
LANES = 16
C = 40
EPT = E // (NC * NS)
NCHUNK = EPT // C
ROW_BLK = 1000


def _bf16_bits(y):
    yb = y.astype(jnp.bfloat16).astype(jnp.float32)
    return lax.shift_right_logical(lax.bitcast_convert_type(yb, jnp.int32), 16)


def _mm_body(x_ref, wk, bk, wqe, bqe, wqo, bqo, wve, bve, wvo, bvo, ws, bs,
             k_out, qv_out, b_out):
    xb = x_ref[...]
    k_out[...] = -(jnp.dot(xb, wk[...], preferred_element_type=jnp.float32)
                   + bk[...])
    qe = jnp.dot(xb, wqe[...], preferred_element_type=jnp.float32) + bqe[...]
    qo = jnp.dot(xb, wqo[...], preferred_element_type=jnp.float32) + bqo[...]
    ve = jnp.dot(xb, wve[...], preferred_element_type=jnp.float32) + bve[...]
    vo = jnp.dot(xb, wvo[...], preferred_element_type=jnp.float32) + bvo[...]
    wq = jnp.bitwise_or(_bf16_bits(qe), lax.shift_left(_bf16_bits(qo), 16))
    wv = jnp.bitwise_or(_bf16_bits(ve), lax.shift_left(_bf16_bits(vo), 16))
    qv_out[...] = jnp.concatenate([wq, wv], axis=1)
    b_out[...] = jnp.dot(xb, ws[...], preferred_element_type=jnp.float32) + bs[...]


def _stage1(x, Wk, bk, Wq, bq, Wv, bv, Ws, bias):
    nblk = N // ROW_BLK
    row_spec = pl.BlockSpec((ROW_BLK, D), lambda i: (i, 0))
    w_spec = pl.BlockSpec((D, D), lambda i: (0, 0))
    h_spec = pl.BlockSpec((D, D // 2), lambda i: (0, 0))
    b_spec = pl.BlockSpec((1, D), lambda i: (0, 0))
    bh_spec = pl.BlockSpec((1, D // 2), lambda i: (0, 0))
    out_qv = jax.ShapeDtypeStruct((N, D), jnp.int32)
    out32 = jax.ShapeDtypeStruct((N, D), jnp.float32)
    lo = jnp.asarray([32 * i + r for i in range(D // 32) for r in range(16)],
                     dtype=jnp.int32)
    hi = lo + 16
    return pl.pallas_call(
        _mm_body,
        grid=(nblk,),
        in_specs=[row_spec, w_spec, b_spec, h_spec, bh_spec, h_spec, bh_spec,
                  h_spec, bh_spec, h_spec, bh_spec, w_spec, b_spec],
        out_specs=[row_spec, row_spec, row_spec],
        out_shape=[out32, out_qv, out32],
    )(x, Wk, bk.reshape(1, D),
      Wq[:, lo], bq[lo].reshape(1, D // 2), Wq[:, hi], bq[hi].reshape(1, D // 2),
      Wv[:, lo], bv[lo].reshape(1, D // 2), Wv[:, hi], bv[hi].reshape(1, D // 2),
      Ws, bias.reshape(1, D))


def _sc_body(src_hbm, dst_hbm, k_hbm, qv_hbm, base_hbm, zero_hbm,
             p0_hbm, p1_hbm,
             di0, si0, ds0, kd0, qv0, mg0, di1, si1, ds1, kd1, qv1, mg1, agg,
             semi0, semk0, semq0, sems0, semi1, semk1, semq1, sems1):
    cid = lax.axis_index("c")
    sid = lax.axis_index("s")
    wid = cid * NS + sid

    @pl.when(sid == 0)
    def _():
        @pl.when(cid == 0)
        def _():
            pltpu.sync_copy(base_hbm, agg)

        @pl.when(cid == 1)
        def _():
            pltpu.sync_copy(zero_hbm, agg)

    plsc.subcore_barrier()

    bufs = ((di0, si0, ds0, kd0, qv0, mg0, semi0, semk0, semq0, sems0),
            (di1, si1, ds1, kd1, qv1, mg1, semi1, semk1, semq1, sems1))

    def idx_start(b, t):
        di, si, dsc, kd, qv, mg, smi, sk, sq, ss = bufs[b]
        pltpu.async_copy(dst_hbm.at[wid, pl.ds(t, 1)], di, smi)
        pltpu.async_copy(dst_hbm.at[wid, pl.ds(t, 1)], dsc, smi)
        pltpu.async_copy(src_hbm.at[wid, pl.ds(t, 1)], si, smi)

    def idx_wait(b, t):
        di, si, dsc, kd, qv, mg, smi, sk, sq, ss = bufs[b]
        pltpu.make_async_copy(dst_hbm.at[wid, pl.ds(t, 1)], di, smi).wait()
        pltpu.make_async_copy(dst_hbm.at[wid, pl.ds(t, 1)], dsc, smi).wait()
        pltpu.make_async_copy(src_hbm.at[wid, pl.ds(t, 1)], si, smi).wait()

    def start_g(b, t):
        di, si, dsc, kd, qv, mg, smi, sk, sq, ss = bufs[b]
        idx_wait(b, t)
        pltpu.async_copy(k_hbm.at[di.at[0]], kd, sk)
        pltpu.async_copy(qv_hbm.at[si.at[0]], qv, sq)

    def scat_wait(b):
        di, si, dsc, kd, qv, mg, smi, sk, sq, ss = bufs[b]
        pltpu.make_async_copy(mg, agg.at[dsc.at[0]], ss).wait()

    def finish(b, t):
        di, si, dsc, kd, qv, mg, smi, sk, sq, ss = bufs[b]
        pltpu.make_async_copy(k_hbm.at[di.at[0]], kd, sk).wait()
        pltpu.make_async_copy(qv_hbm.at[si.at[0]], qv, sq).wait()

        @pl.when(t >= 2)
        def _():
            scat_wait(b)

        hi_mask = jnp.full((LANES,), -65536, dtype=jnp.int32)
        sh16 = jnp.full((LANES,), 16, dtype=jnp.int32)
        one = jnp.full((LANES,), 1.0, dtype=jnp.float32)

        def _halves(w):
            lo = lax.bitcast_convert_type(lax.shift_left(w, sh16), jnp.float32)
            hi = lax.bitcast_convert_type(lax.bitwise_and(w, hi_mask), jnp.float32)
            return lo, hi

        @plsc.parallel_loop(0, C, unroll=4)
        def _(j):
            for i in range(D // 32):
                qa, qb = _halves(qv[j, pl.ds(i * LANES, LANES)])
                va, vb = _halves(qv[j, pl.ds(D // 2 + i * LANES, LANES)])
                ka = kd[j, pl.ds(i * 32, LANES)]
                kb = kd[j, pl.ds(i * 32 + LANES, LANES)]
                mg[j, pl.ds(i * 32, LANES)] = va / (one + jnp.exp(ka - qa))
                mg[j, pl.ds(i * 32 + LANES, LANES)] = vb / (one + jnp.exp(kb - qb))
        pltpu.async_copy(mg, agg.at[dsc.at[0]], ss, add=True)

        @pl.when(t + 2 < NCHUNK)
        def _():
            idx_start(b, t + 2)

    idx_start(0, 0)
    idx_start(1, 1)
    start_g(0, 0)

    def pair_body(it, carry):
        c0 = it * 2
        c1 = c0 + 1
        start_g(1, c1)
        finish(0, c0)

        @pl.when(c1 + 1 < NCHUNK)
        def _():
            start_g(0, c1 + 1)

        finish(1, c1)
        return carry

    lax.fori_loop(0, NCHUNK // 2, pair_body, 0)

    scat_wait(0)
    scat_wait(1)

    plsc.subcore_barrier()

    rows_pt = 624
    tail = N - (NS - 1) * rows_pt
    roff = pl.multiple_of(sid * rows_pt, 8)

    @pl.when(cid == 0)
    def _():
        @pl.when(sid < NS - 1)
        def _():
            pltpu.sync_copy(agg.at[pl.ds(roff, rows_pt)],
                            p0_hbm.at[pl.ds(roff, rows_pt)])

        @pl.when(sid == NS - 1)
        def _():
            pltpu.sync_copy(agg.at[pl.ds((NS - 1) * rows_pt, tail)],
                            p0_hbm.at[pl.ds((NS - 1) * rows_pt, tail)])

    @pl.when(cid == 1)
    def _():
        @pl.when(sid < NS - 1)
        def _():
            pltpu.sync_copy(agg.at[pl.ds(roff, rows_pt)],
                            p1_hbm.at[pl.ds(roff, rows_pt)])

        @pl.when(sid == NS - 1)
        def _():
            pltpu.sync_copy(agg.at[pl.ds((NS - 1) * rows_pt, tail)],
                            p1_hbm.at[pl.ds((NS - 1) * rows_pt, tail)])


@functools.lru_cache(maxsize=1)
def _sc_edges():
    return pl.kernel(
        _sc_body,
        out_type=[jax.ShapeDtypeStruct((N, D), jnp.float32)] * 2,
        mesh=plsc.VectorSubcoreMesh(core_axis_name="c", subcore_axis_name="s",
                                    num_cores=NC, num_subcores=NS),
        scratch_types=[
            pltpu.VMEM((1, C), jnp.int32),
            pltpu.VMEM((1, C), jnp.int32),
            pltpu.VMEM((1, C), jnp.int32),
            pltpu.VMEM((C, D), jnp.float32),
            pltpu.VMEM((C, D), jnp.int32),
            pltpu.VMEM((C, D), jnp.float32),
            pltpu.VMEM((1, C), jnp.int32),
            pltpu.VMEM((1, C), jnp.int32),
            pltpu.VMEM((1, C), jnp.int32),
            pltpu.VMEM((C, D), jnp.float32),
            pltpu.VMEM((C, D), jnp.int32),
            pltpu.VMEM((C, D), jnp.float32),
            pltpu.VMEM_SHARED((N, D), jnp.float32),
            pltpu.SemaphoreType.DMA,
            pltpu.SemaphoreType.DMA,
            pltpu.SemaphoreType.DMA,
            pltpu.SemaphoreType.DMA,
            pltpu.SemaphoreType.DMA,
            pltpu.SemaphoreType.DMA,
            pltpu.SemaphoreType.DMA,
            pltpu.SemaphoreType.DMA,
        ],
    )


def _add_body(a_ref, b_ref, o_ref):
    o_ref[...] = a_ref[...] + b_ref[...]


def _stage3(p0, p1):
    row_spec = pl.BlockSpec((ROW_BLK, D), lambda i: (i, 0))
    return pl.pallas_call(
        _add_body,
        grid=(N // ROW_BLK,),
        in_specs=[row_spec, row_spec],
        out_specs=row_spec,
        out_shape=jax.ShapeDtypeStruct((N, D), jnp.float32),
    )(p0, p1)


def kernel(x, edge_index, Wk, bk, Wq, bq, Wv, bv, Ws, bias):
    src = edge_index[0].reshape(NC * NS, NCHUNK, C)
    dst = edge_index[1].reshape(NC * NS, NCHUNK, C)
    kneg, qv32, base = _stage1(x, Wk, bk, Wq, bq, Wv, bv, Ws, bias)
    zeros = jnp.zeros((N, D), dtype=jnp.float32)
    p0, p1 = _sc_edges()(src, dst, kneg, qv32, base, zeros)
    return _stage3(p0, p1)


# speedup vs baseline: 1.0245x; 1.0152x over previous
import functools

import jax
import jax.numpy as jnp
from jax import lax
from jax.experimental import pallas as pl
from jax.experimental.pallas import tpu as pltpu
from jax.experimental.pallas import tpu_sc as plsc

N = 10000
E = 320000
D = 128

NC = 2
NS = 16
LANES = 16
C = 40
EPT = E // (NC * NS)
NCHUNK = EPT // C
ROW_BLK = 1000


def _bf16_bits(y):
    yb = y.astype(jnp.bfloat16).astype(jnp.float32)
    return lax.shift_right_logical(lax.bitcast_convert_type(yb, jnp.int32), 16)


def _mm_body(x_ref, wk, bk, wqe, bqe, wqo, bqo, wve, bve, wvo, bvo, ws, bs,
             k_out, qv_out, b_out):
    xb = x_ref[...]
    k_out[...] = -(jnp.dot(xb, wk[...], preferred_element_type=jnp.float32)
                   + bk[...])
    qe = jnp.dot(xb, wqe[...], preferred_element_type=jnp.float32) + bqe[...]
    qo = jnp.dot(xb, wqo[...], preferred_element_type=jnp.float32) + bqo[...]
    ve = jnp.dot(xb, wve[...], preferred_element_type=jnp.float32) + bve[...]
    vo = jnp.dot(xb, wvo[...], preferred_element_type=jnp.float32) + bvo[...]
    wq = jnp.bitwise_or(_bf16_bits(qe), lax.shift_left(_bf16_bits(qo), 16))
    wv = jnp.bitwise_or(_bf16_bits(ve), lax.shift_left(_bf16_bits(vo), 16))
    qv_out[...] = jnp.concatenate([wq, wv], axis=1)
    b_out[...] = (jnp.dot(xb, ws[...], preferred_element_type=jnp.float32)
                  + bs[...]) * 0.5


def _stage1(x, Wk, bk, Wq, bq, Wv, bv, Ws, bias):
    nblk = N // ROW_BLK
    row_spec = pl.BlockSpec((ROW_BLK, D), lambda i: (i, 0))
    w_spec = pl.BlockSpec((D, D), lambda i: (0, 0))
    h_spec = pl.BlockSpec((D, D // 2), lambda i: (0, 0))
    b_spec = pl.BlockSpec((1, D), lambda i: (0, 0))
    bh_spec = pl.BlockSpec((1, D // 2), lambda i: (0, 0))
    out_qv = jax.ShapeDtypeStruct((N, D), jnp.int32)
    out32 = jax.ShapeDtypeStruct((N, D), jnp.float32)
    lo = jnp.asarray([32 * i + r for i in range(D // 32) for r in range(16)],
                     dtype=jnp.int32)
    hi = lo + 16
    return pl.pallas_call(
        _mm_body,
        grid=(nblk,),
        in_specs=[row_spec, w_spec, b_spec, h_spec, bh_spec, h_spec, bh_spec,
                  h_spec, bh_spec, h_spec, bh_spec, w_spec, b_spec],
        out_specs=[row_spec, row_spec, row_spec],
        out_shape=[out32, out_qv, out32],
    )(x, Wk, bk.reshape(1, D),
      Wq[:, lo], bq[lo].reshape(1, D // 2), Wq[:, hi], bq[hi].reshape(1, D // 2),
      Wv[:, lo], bv[lo].reshape(1, D // 2), Wv[:, hi], bv[hi].reshape(1, D // 2),
      Ws, bias.reshape(1, D))


def _sc_body(src_hbm, dst_hbm, k_hbm, qv_hbm, base_hbm,
             p0_hbm, p1_hbm,
             di0, si0, ds0, kd0, qv0, mg0, di1, si1, ds1, kd1, qv1, mg1, agg,
             semi0, semk0, semq0, sems0, semi1, semk1, semq1, sems1):
    cid = lax.axis_index("c")
    sid = lax.axis_index("s")
    wid = cid * NS + sid

    @pl.when(sid == 0)
    def _():
        pltpu.sync_copy(base_hbm, agg)

    plsc.subcore_barrier()

    bufs = ((di0, si0, ds0, kd0, qv0, mg0, semi0, semk0, semq0, sems0),
            (di1, si1, ds1, kd1, qv1, mg1, semi1, semk1, semq1, sems1))

    def idx_start(b, t):
        di, si, dsc, kd, qv, mg, smi, sk, sq, ss = bufs[b]
        pltpu.async_copy(dst_hbm.at[wid, pl.ds(t, 1)], di, smi)
        pltpu.async_copy(dst_hbm.at[wid, pl.ds(t, 1)], dsc, smi)
        pltpu.async_copy(src_hbm.at[wid, pl.ds(t, 1)], si, smi)

    def idx_wait(b, t):
        di, si, dsc, kd, qv, mg, smi, sk, sq, ss = bufs[b]
        pltpu.make_async_copy(dst_hbm.at[wid, pl.ds(t, 1)], di, smi).wait()
        pltpu.make_async_copy(dst_hbm.at[wid, pl.ds(t, 1)], dsc, smi).wait()
        pltpu.make_async_copy(src_hbm.at[wid, pl.ds(t, 1)], si, smi).wait()

    def start_g(b, t):
        di, si, dsc, kd, qv, mg, smi, sk, sq, ss = bufs[b]
        idx_wait(b, t)
        pltpu.async_copy(k_hbm.at[di.at[0]], kd, sk)
        pltpu.async_copy(qv_hbm.at[si.at[0]], qv, sq)

    def scat_wait(b):
        di, si, dsc, kd, qv, mg, smi, sk, sq, ss = bufs[b]
        pltpu.make_async_copy(mg, agg.at[dsc.at[0]], ss).wait()

    def finish(b, t):
        di, si, dsc, kd, qv, mg, smi, sk, sq, ss = bufs[b]
        pltpu.make_async_copy(k_hbm.at[di.at[0]], kd, sk).wait()
        pltpu.make_async_copy(qv_hbm.at[si.at[0]], qv, sq).wait()

        @pl.when(t >= 2)
        def _():
            scat_wait(b)

        hi_mask = jnp.full((LANES,), -65536, dtype=jnp.int32)
        sh16 = jnp.full((LANES,), 16, dtype=jnp.int32)
        one = jnp.full((LANES,), 1.0, dtype=jnp.float32)

        def _halves(w):
            lo = lax.bitcast_convert_type(lax.shift_left(w, sh16), jnp.float32)
            hi = lax.bitcast_convert_type(lax.bitwise_and(w, hi_mask), jnp.float32)
            return lo, hi

        def row_body(j, rcarry):
            for i in range(D // 32):
                qa, qb = _halves(qv[j, pl.ds(i * LANES, LANES)])
                va, vb = _halves(qv[j, pl.ds(D // 2 + i * LANES, LANES)])
                ka = kd[j, pl.ds(i * 32, LANES)]
                kb = kd[j, pl.ds(i * 32 + LANES, LANES)]
                mg[j, pl.ds(i * 32, LANES)] = va / (one + jnp.exp(ka - qa))
                mg[j, pl.ds(i * 32 + LANES, LANES)] = vb / (one + jnp.exp(kb - qb))
            return rcarry

        lax.fori_loop(0, C, row_body, 0)
        pltpu.async_copy(mg, agg.at[dsc.at[0]], ss, add=True)

        @pl.when(t + 2 < NCHUNK)
        def _():
            idx_start(b, t + 2)

    idx_start(0, 0)
    idx_start(1, 1)
    start_g(0, 0)

    def pair_body(it, carry):
        c0 = it * 2
        c1 = c0 + 1
        start_g(1, c1)
        finish(0, c0)

        @pl.when(c1 + 1 < NCHUNK)
        def _():
            start_g(0, c1 + 1)

        finish(1, c1)
        return carry

    lax.fori_loop(0, NCHUNK // 2, pair_body, 0)

    scat_wait(0)
    scat_wait(1)

    plsc.subcore_barrier()

    rows_pt = 624
    tail = N - (NS - 1) * rows_pt
    roff = pl.multiple_of(sid * rows_pt, 8)

    @pl.when(cid == 0)
    def _():
        @pl.when(sid < NS - 1)
        def _():
            pltpu.sync_copy(agg.at[pl.ds(roff, rows_pt)],
                            p0_hbm.at[pl.ds(roff, rows_pt)])

        @pl.when(sid == NS - 1)
        def _():
            pltpu.sync_copy(agg.at[pl.ds((NS - 1) * rows_pt, tail)],
                            p0_hbm.at[pl.ds((NS - 1) * rows_pt, tail)])

    @pl.when(cid == 1)
    def _():
        @pl.when(sid < NS - 1)
        def _():
            pltpu.sync_copy(agg.at[pl.ds(roff, rows_pt)],
                            p1_hbm.at[pl.ds(roff, rows_pt)])

        @pl.when(sid == NS - 1)
        def _():
            pltpu.sync_copy(agg.at[pl.ds((NS - 1) * rows_pt, tail)],
                            p1_hbm.at[pl.ds((NS - 1) * rows_pt, tail)])


@functools.lru_cache(maxsize=1)
def _sc_edges():
    return pl.kernel(
        _sc_body,
        out_type=[jax.ShapeDtypeStruct((N, D), jnp.float32)] * 2,
        mesh=plsc.VectorSubcoreMesh(core_axis_name="c", subcore_axis_name="s",
                                    num_cores=NC, num_subcores=NS),
        scratch_types=[
            pltpu.VMEM((1, C), jnp.int32),
            pltpu.VMEM((1, C), jnp.int32),
            pltpu.VMEM((1, C), jnp.int32),
            pltpu.VMEM((C, D), jnp.float32),
            pltpu.VMEM((C, D), jnp.int32),
            pltpu.VMEM((C, D), jnp.float32),
            pltpu.VMEM((1, C), jnp.int32),
            pltpu.VMEM((1, C), jnp.int32),
            pltpu.VMEM((1, C), jnp.int32),
            pltpu.VMEM((C, D), jnp.float32),
            pltpu.VMEM((C, D), jnp.int32),
            pltpu.VMEM((C, D), jnp.float32),
            pltpu.VMEM_SHARED((N, D), jnp.float32),
            pltpu.SemaphoreType.DMA,
            pltpu.SemaphoreType.DMA,
            pltpu.SemaphoreType.DMA,
            pltpu.SemaphoreType.DMA,
            pltpu.SemaphoreType.DMA,
            pltpu.SemaphoreType.DMA,
            pltpu.SemaphoreType.DMA,
            pltpu.SemaphoreType.DMA,
        ],
    )


def _add_body(a_ref, b_ref, o_ref):
    o_ref[...] = a_ref[...] + b_ref[...]


def _stage3(p0, p1):
    row_spec = pl.BlockSpec((ROW_BLK, D), lambda i: (i, 0))
    return pl.pallas_call(
        _add_body,
        grid=(N // ROW_BLK,),
        in_specs=[row_spec, row_spec],
        out_specs=row_spec,
        out_shape=jax.ShapeDtypeStruct((N, D), jnp.float32),
    )(p0, p1)


def kernel(x, edge_index, Wk, bk, Wq, bq, Wv, bv, Ws, bias):
    src = edge_index[0].reshape(NC * NS, NCHUNK, C)
    dst = edge_index[1].reshape(NC * NS, NCHUNK, C)
    kneg, qv32, base_half = _stage1(x, Wk, bk, Wq, bq, Wv, bv, Ws, bias)
    p0, p1 = _sc_edges()(src, dst, kneg, qv32, base_half)
    return _stage3(p0, p1)


# speedup vs baseline: 1.0418x; 1.0169x over previous
import functools

import jax
import jax.numpy as jnp
from jax import lax
from jax.experimental import pallas as pl
from jax.experimental.pallas import tpu as pltpu
from jax.experimental.pallas import tpu_sc as plsc

N = 10000
E = 320000
D = 128

NC = 2
NS = 16
LANES = 16
C = 40
EPT = E // (NC * NS)
NCHUNK = EPT // C
ROW_BLK = 2000


def _bf16_bits(y):
    yb = y.astype(jnp.bfloat16).astype(jnp.float32)
    return lax.shift_right_logical(lax.bitcast_convert_type(yb, jnp.int32), 16)


def _mm_body(x_ref, wk, bk, wqe, bqe, wqo, bqo, wve, bve, wvo, bvo, ws, bs,
             k_out, qv_out, b_out):
    xb = x_ref[...]
    k_out[...] = -(jnp.dot(xb, wk[...], preferred_element_type=jnp.float32)
                   + bk[...])
    qe = jnp.dot(xb, wqe[...], preferred_element_type=jnp.float32) + bqe[...]
    qo = jnp.dot(xb, wqo[...], preferred_element_type=jnp.float32) + bqo[...]
    ve = jnp.dot(xb, wve[...], preferred_element_type=jnp.float32) + bve[...]
    vo = jnp.dot(xb, wvo[...], preferred_element_type=jnp.float32) + bvo[...]
    wq = jnp.bitwise_or(_bf16_bits(qe), lax.shift_left(_bf16_bits(qo), 16))
    wv = jnp.bitwise_or(_bf16_bits(ve), lax.shift_left(_bf16_bits(vo), 16))
    qv_out[...] = jnp.concatenate([wq, wv], axis=1)
    b_out[...] = (jnp.dot(xb, ws[...], preferred_element_type=jnp.float32)
                  + bs[...]) * 0.5


def _stage1(x, Wk, bk, Wq, bq, Wv, bv, Ws, bias):
    nblk = N // ROW_BLK
    row_spec = pl.BlockSpec((ROW_BLK, D), lambda i: (i, 0))
    w_spec = pl.BlockSpec((D, D), lambda i: (0, 0))
    h_spec = pl.BlockSpec((D, D // 2), lambda i: (0, 0))
    b_spec = pl.BlockSpec((1, D), lambda i: (0, 0))
    bh_spec = pl.BlockSpec((1, D // 2), lambda i: (0, 0))
    out_qv = jax.ShapeDtypeStruct((N, D), jnp.int32)
    out32 = jax.ShapeDtypeStruct((N, D), jnp.float32)
    lo = jnp.asarray([32 * i + r for i in range(D // 32) for r in range(16)],
                     dtype=jnp.int32)
    hi = lo + 16
    return pl.pallas_call(
        _mm_body,
        grid=(nblk,),
        in_specs=[row_spec, w_spec, b_spec, h_spec, bh_spec, h_spec, bh_spec,
                  h_spec, bh_spec, h_spec, bh_spec, w_spec, b_spec],
        out_specs=[row_spec, row_spec, row_spec],
        out_shape=[out32, out_qv, out32],
    )(x, Wk, bk.reshape(1, D),
      Wq[:, lo], bq[lo].reshape(1, D // 2), Wq[:, hi], bq[hi].reshape(1, D // 2),
      Wv[:, lo], bv[lo].reshape(1, D // 2), Wv[:, hi], bv[hi].reshape(1, D // 2),
      Ws, bias.reshape(1, D))


def _sc_body(src_hbm, dst_hbm, k_hbm, qv_hbm, base_hbm,
             p0_hbm, p1_hbm,
             di0, si0, ds0, kd0, qv0, mg0, di1, si1, ds1, kd1, qv1, mg1, agg,
             semi0, semk0, semq0, sems0, semi1, semk1, semq1, sems1):
    cid = lax.axis_index("c")
    sid = lax.axis_index("s")
    wid = cid * NS + sid

    @pl.when(sid == 0)
    def _():
        pltpu.sync_copy(base_hbm, agg)

    plsc.subcore_barrier()

    bufs = ((di0, si0, ds0, kd0, qv0, mg0, semi0, semk0, semq0, sems0),
            (di1, si1, ds1, kd1, qv1, mg1, semi1, semk1, semq1, sems1))

    def idx_start(b, t):
        di, si, dsc, kd, qv, mg, smi, sk, sq, ss = bufs[b]
        pltpu.async_copy(dst_hbm.at[wid, pl.ds(t, 1)], di, smi)
        pltpu.async_copy(dst_hbm.at[wid, pl.ds(t, 1)], dsc, smi)
        pltpu.async_copy(src_hbm.at[wid, pl.ds(t, 1)], si, smi)

    def idx_wait(b, t):
        di, si, dsc, kd, qv, mg, smi, sk, sq, ss = bufs[b]
        pltpu.make_async_copy(dst_hbm.at[wid, pl.ds(t, 1)], di, smi).wait()
        pltpu.make_async_copy(dst_hbm.at[wid, pl.ds(t, 1)], dsc, smi).wait()
        pltpu.make_async_copy(src_hbm.at[wid, pl.ds(t, 1)], si, smi).wait()

    def start_g(b, t):
        di, si, dsc, kd, qv, mg, smi, sk, sq, ss = bufs[b]
        idx_wait(b, t)
        pltpu.async_copy(k_hbm.at[di.at[0]], kd, sk)
        pltpu.async_copy(qv_hbm.at[si.at[0]], qv, sq)

    def scat_wait(b):
        di, si, dsc, kd, qv, mg, smi, sk, sq, ss = bufs[b]
        pltpu.make_async_copy(mg, agg.at[dsc.at[0]], ss).wait()

    def finish(b, t):
        di, si, dsc, kd, qv, mg, smi, sk, sq, ss = bufs[b]
        pltpu.make_async_copy(k_hbm.at[di.at[0]], kd, sk).wait()
        pltpu.make_async_copy(qv_hbm.at[si.at[0]], qv, sq).wait()

        @pl.when(t >= 2)
        def _():
            scat_wait(b)

        hi_mask = jnp.full((LANES,), -65536, dtype=jnp.int32)
        sh16 = jnp.full((LANES,), 16, dtype=jnp.int32)
        one = jnp.full((LANES,), 1.0, dtype=jnp.float32)

        def _halves(w):
            lo = lax.bitcast_convert_type(lax.shift_left(w, sh16), jnp.float32)
            hi = lax.bitcast_convert_type(lax.bitwise_and(w, hi_mask), jnp.float32)
            return lo, hi

        def row_body(j, rcarry):
            for i in range(D // 32):
                qa, qb = _halves(qv[j, pl.ds(i * LANES, LANES)])
                va, vb = _halves(qv[j, pl.ds(D // 2 + i * LANES, LANES)])
                ka = kd[j, pl.ds(i * 32, LANES)]
                kb = kd[j, pl.ds(i * 32 + LANES, LANES)]
                mg[j, pl.ds(i * 32, LANES)] = va / (one + jnp.exp(ka - qa))
                mg[j, pl.ds(i * 32 + LANES, LANES)] = vb / (one + jnp.exp(kb - qb))
            return rcarry

        lax.fori_loop(0, C, row_body, 0)
        pltpu.async_copy(mg, agg.at[dsc.at[0]], ss, add=True)

        @pl.when(t + 2 < NCHUNK)
        def _():
            idx_start(b, t + 2)

    idx_start(0, 0)
    idx_start(1, 1)
    start_g(0, 0)

    def pair_body(it, carry):
        c0 = it * 2
        c1 = c0 + 1
        start_g(1, c1)
        finish(0, c0)

        @pl.when(c1 + 1 < NCHUNK)
        def _():
            start_g(0, c1 + 1)

        finish(1, c1)
        return carry

    lax.fori_loop(0, NCHUNK // 2, pair_body, 0)

    scat_wait(0)
    scat_wait(1)

    plsc.subcore_barrier()

    rows_pt = 624
    tail = N - (NS - 1) * rows_pt
    roff = pl.multiple_of(sid * rows_pt, 8)

    @pl.when(cid == 0)
    def _():
        @pl.when(sid < NS - 1)
        def _():
            pltpu.sync_copy(agg.at[pl.ds(roff, rows_pt)],
                            p0_hbm.at[pl.ds(roff, rows_pt)])

        @pl.when(sid == NS - 1)
        def _():
            pltpu.sync_copy(agg.at[pl.ds((NS - 1) * rows_pt, tail)],
                            p0_hbm.at[pl.ds((NS - 1) * rows_pt, tail)])

    @pl.when(cid == 1)
    def _():
        @pl.when(sid < NS - 1)
        def _():
            pltpu.sync_copy(agg.at[pl.ds(roff, rows_pt)],
                            p1_hbm.at[pl.ds(roff, rows_pt)])

        @pl.when(sid == NS - 1)
        def _():
            pltpu.sync_copy(agg.at[pl.ds((NS - 1) * rows_pt, tail)],
                            p1_hbm.at[pl.ds((NS - 1) * rows_pt, tail)])


@functools.lru_cache(maxsize=1)
def _sc_edges():
    return pl.kernel(
        _sc_body,
        out_type=[jax.ShapeDtypeStruct((N, D), jnp.float32)] * 2,
        mesh=plsc.VectorSubcoreMesh(core_axis_name="c", subcore_axis_name="s",
                                    num_cores=NC, num_subcores=NS),
        scratch_types=[
            pltpu.VMEM((1, C), jnp.int32),
            pltpu.VMEM((1, C), jnp.int32),
            pltpu.VMEM((1, C), jnp.int32),
            pltpu.VMEM((C, D), jnp.float32),
            pltpu.VMEM((C, D), jnp.int32),
            pltpu.VMEM((C, D), jnp.float32),
            pltpu.VMEM((1, C), jnp.int32),
            pltpu.VMEM((1, C), jnp.int32),
            pltpu.VMEM((1, C), jnp.int32),
            pltpu.VMEM((C, D), jnp.float32),
            pltpu.VMEM((C, D), jnp.int32),
            pltpu.VMEM((C, D), jnp.float32),
            pltpu.VMEM_SHARED((N, D), jnp.float32),
            pltpu.SemaphoreType.DMA,
            pltpu.SemaphoreType.DMA,
            pltpu.SemaphoreType.DMA,
            pltpu.SemaphoreType.DMA,
            pltpu.SemaphoreType.DMA,
            pltpu.SemaphoreType.DMA,
            pltpu.SemaphoreType.DMA,
            pltpu.SemaphoreType.DMA,
        ],
    )


def _add_body(a_ref, b_ref, o_ref):
    o_ref[...] = a_ref[...] + b_ref[...]


def _stage3(p0, p1):
    row_spec = pl.BlockSpec((ROW_BLK, D), lambda i: (i, 0))
    return pl.pallas_call(
        _add_body,
        grid=(N // ROW_BLK,),
        in_specs=[row_spec, row_spec],
        out_specs=row_spec,
        out_shape=jax.ShapeDtypeStruct((N, D), jnp.float32),
    )(p0, p1)


def kernel(x, edge_index, Wk, bk, Wq, bq, Wv, bv, Ws, bias):
    src = edge_index[0].reshape(NC * NS, NCHUNK, C)
    dst = edge_index[1].reshape(NC * NS, NCHUNK, C)
    kneg, qv32, base_half = _stage1(x, Wk, bk, Wq, bq, Wv, bv, Ws, bias)
    p0, p1 = _sc_edges()(src, dst, kneg, qv32, base_half)
    return _stage3(p0, p1)
